# baseline (speedup 1.0000x reference)
import functools

import jax
import jax.numpy as jnp
from jax import lax
from jax.experimental import pallas as pl
from jax.experimental.pallas import tpu as pltpu
from jax.experimental.pallas import tpu_sc as plsc

N = 10000
E = 160000
D_IN = 256
D_HID = 512
D_HID2 = 256
NC = 2
NT = 16
B = 128
EP = NC * NT * 40 * B
NB_PROP = EP // (NT * B)
NB_DEG = EP // (NC * NT * B)
ACC_ROWS = N + 16
S0 = 632
SLAST = N - (NT - 1) * S0
BM = 2000

_mesh = plsc.VectorSubcoreMesh(core_axis_name="c", subcore_axis_name="s")


B2 = 64
NB2 = EP // (NT * B2)
CB = 40
NCHUNK = NB2 // CB
NBUF = 4


@functools.partial(
    pl.kernel,
    out_type=jax.ShapeDtypeStruct((NC, N, 128), jnp.float32),
    mesh=_mesh,
    scratch_types=[
        pltpu.VMEM_SHARED((ACC_ROWS, 128), jnp.float32),
        pltpu.VMEM((CB, B2), jnp.int32),
        pltpu.VMEM((CB, B2), jnp.int32),
        pltpu.VMEM((NBUF, B2, 128), jnp.float32),
        pltpu.SemaphoreType.DMA((NBUF,)),
        pltpu.SemaphoreType.DMA((NBUF,)),
    ],
)
def _sc_prop(table_hbm, src_hbm, dstidx_hbm, out_hbm,
             acc, src_v, dst_v, bufs, sems, ssems):
    c = lax.axis_index("c")
    s = lax.axis_index("s")

    @pl.when(s < NT - 1)
    def _():
        pltpu.sync_copy(table_hbm.at[pl.ds(c * N + s * S0, S0)],
                        acc.at[pl.ds(s * S0, S0)])

    @pl.when(s == NT - 1)
    def _():
        pltpu.sync_copy(table_hbm.at[pl.ds(c * N + (NT - 1) * S0, SLAST)],
                        acc.at[pl.ds((NT - 1) * S0, SLAST)])

    plsc.subcore_barrier()

    def gather(b, j):
        pltpu.async_copy(table_hbm.at[src_v.at[b]], bufs.at[j], sems.at[j])

    coff = (c * N).astype(jnp.int32)

    def chunk(k, _):
        pltpu.sync_copy(src_hbm.at[s, pl.ds(k * CB, CB)], src_v)
        pltpu.sync_copy(dstidx_hbm.at[s, pl.ds(k * CB, CB)], dst_v)

        def shift(i, _):
            r = i // 4
            q = (i % 4) * 16
            src_v[r, pl.ds(q, 16)] = src_v[r, pl.ds(q, 16)] + coff
            return 0

        lax.fori_loop(0, CB * 4, shift, 0)
        for j in range(NBUF - 1):
            gather(j, j)

        def group(g, _):
            for j in range(NBUF):
                b = NBUF * g + j
                jj = (j + NBUF - 1) % NBUF
                pltpu.make_async_copy(table_hbm.at[src_v.at[b]],
                                      bufs.at[j], sems.at[j]).wait()
                pltpu.async_copy(bufs.at[j], acc.at[dst_v.at[b]],
                                 ssems.at[j], add=True)

                @pl.when(b + NBUF - 1 < CB)
                def _(b=b, j=j, jj=jj):
                    @pl.when(b >= 1)
                    def _():
                        pltpu.make_async_copy(
                            bufs.at[jj], acc.at[dst_v.at[b - 1]],
                            ssems.at[jj]).wait()

                    gather(b + NBUF - 1, jj)
            return 0

        lax.fori_loop(0, CB // NBUF, group, 0)
        for j in range(NBUF):
            b = CB - NBUF + j
            pltpu.make_async_copy(bufs.at[j], acc.at[dst_v.at[b]],
                                  ssems.at[j]).wait()
        return 0

    lax.fori_loop(0, NCHUNK, chunk, 0)
    plsc.subcore_barrier()

    @pl.when(s < NT - 1)
    def _():
        pltpu.sync_copy(acc.at[pl.ds(s * S0, S0)],
                        out_hbm.at[c, pl.ds(s * S0, S0)])

    @pl.when(s == NT - 1)
    def _():
        pltpu.sync_copy(acc.at[pl.ds((NT - 1) * S0, SLAST)],
                        out_hbm.at[c, pl.ds((NT - 1) * S0, SLAST)])


NB_CNT = EP // (NC * NT * B)
_Z_CHUNKS = ((128, 128, 128, 128, 120), (128, 128, 128, 128, 8))


@functools.partial(
    pl.kernel,
    out_type=jax.ShapeDtypeStruct((NC, N, 128), jnp.float32),
    mesh=_mesh,
    scratch_types=[
        pltpu.VMEM_SHARED((ACC_ROWS, 128), jnp.float32),
        pltpu.VMEM((NB_CNT, B), jnp.int32),
        pltpu.VMEM((B, 128), jnp.float32),
        pltpu.VMEM((B, 128), jnp.float32),
    ],
)
def _sc_count(dst_hbm, cnt_hbm, acc, dst_v, ones_v, z_v):
    c = lax.axis_index("c")
    s = lax.axis_index("s")
    pltpu.sync_copy(dst_hbm.at[c, s], dst_v)

    def fill(i, _):
        r = i // 8
        j = i % 8
        ones_v[r, pl.ds(j * 16, 16)] = jnp.ones((16,), jnp.float32)
        z_v[r, pl.ds(j * 16, 16)] = jnp.zeros((16,), jnp.float32)
        return 0

    lax.fori_loop(0, B * 8, fill, 0)

    for last, sizes in ((False, _Z_CHUNKS[0]), (True, _Z_CHUNKS[1])):

        @pl.when((s == NT - 1) if last else (s < NT - 1))
        def _(sizes=sizes):
            off = 0
            for sz in sizes:
                pltpu.sync_copy(z_v.at[pl.ds(0, sz)],
                                acc.at[pl.ds(s * S0 + off, sz)])
                off += sz

    plsc.subcore_barrier()

    def body(b, _):
        pltpu.sync_copy(ones_v, acc.at[dst_v.at[b]], add=True)
        return 0

    lax.fori_loop(0, NB_CNT, body, 0)
    plsc.subcore_barrier()

    @pl.when(s < NT - 1)
    def _():
        pltpu.sync_copy(acc.at[pl.ds(s * S0, S0)],
                        cnt_hbm.at[c, pl.ds(s * S0, S0)])

    @pl.when(s == NT - 1)
    def _():
        pltpu.sync_copy(acc.at[pl.ds((NT - 1) * S0, SLAST)],
                        cnt_hbm.at[c, pl.ds((NT - 1) * S0, SLAST)])


def _prep_body(x_ref, cnt_ref, o_ref):
    d = lax.rsqrt(cnt_ref[0] + cnt_ref[1] + 1.0)
    o_ref[0] = x_ref[...] * d


_prep_call = pl.pallas_call(
    _prep_body,
    grid=(2,),
    in_specs=[
        pl.BlockSpec((N, 128), lambda c: (0, c)),
        pl.BlockSpec((NC, N, 128), lambda c: (0, 0, 0)),
    ],
    out_specs=pl.BlockSpec((1, N, 128), lambda c: (c, 0, 0)),
    out_shape=jax.ShapeDtypeStruct((NC, N, 128), jnp.float32),
)


def _mm1_body(p_ref, cnt_ref, w1_ref, b1_ref, w2_ref, o_ref):
    d = lax.rsqrt(cnt_ref[0] + cnt_ref[1] + 1.0)
    p0 = p_ref[0] * d
    p1 = p_ref[1] * d
    h = jnp.dot(p0, w1_ref[:128, :], preferred_element_type=jnp.float32)
    h = h + jnp.dot(p1, w1_ref[128:, :], preferred_element_type=jnp.float32)
    h = jnp.maximum(h + b1_ref[...], 0.0)
    t = jnp.dot(h, w2_ref[...], preferred_element_type=jnp.float32)
    o_ref[0] = t[:, :128] * d
    o_ref[1] = t[:, 128:] * d


_mm1_call = pl.pallas_call(
    _mm1_body,
    grid=(N // BM,),
    in_specs=[
        pl.BlockSpec((NC, BM, 128), lambda i: (0, i, 0)),
        pl.BlockSpec((NC, BM, 128), lambda i: (0, i, 0)),
        pl.BlockSpec((D_IN, D_HID), lambda i: (0, 0)),
        pl.BlockSpec((1, D_HID), lambda i: (0, 0)),
        pl.BlockSpec((D_HID, D_HID2), lambda i: (0, 0)),
    ],
    out_specs=pl.BlockSpec((NC, BM, 128), lambda i: (0, i, 0)),
    out_shape=jax.ShapeDtypeStruct((NC, N, 128), jnp.float32),
)


def _fin_body(p_ref, cnt_ref, b2_ref, wo_ref, bo_ref, o_ref):
    d = lax.rsqrt(cnt_ref[0] + cnt_ref[1] + 1.0)
    b2 = b2_ref[...]
    h0 = jnp.maximum(p_ref[0] * d + b2[:, :128], 0.0)
    h1 = jnp.maximum(p_ref[1] * d + b2[:, 128:], 0.0)
    o = jnp.dot(h0, wo_ref[:128, :], preferred_element_type=jnp.float32)
    o = o + jnp.dot(h1, wo_ref[128:, :], preferred_element_type=jnp.float32)
    o_ref[...] = o + bo_ref[...]


_fin_call = pl.pallas_call(
    _fin_body,
    grid=(N // BM,),
    in_specs=[
        pl.BlockSpec((NC, BM, 128), lambda i: (0, i, 0)),
        pl.BlockSpec((NC, BM, 128), lambda i: (0, i, 0)),
        pl.BlockSpec((1, D_HID2), lambda i: (0, 0)),
        pl.BlockSpec((D_HID2, 1), lambda i: (0, 0)),
        pl.BlockSpec((1, 1), lambda i: (0, 0)),
    ],
    out_specs=pl.BlockSpec((BM, 1), lambda i: (i, 0)),
    out_shape=jax.ShapeDtypeStruct((N, 1), jnp.float32),
)


def kernel(x, edge_index, W1, b1, W2, b2, Wout, bout):
    src = edge_index[0].astype(jnp.int32)
    dst = edge_index[1].astype(jnp.int32)
    pad = jnp.arange(EP - E, dtype=jnp.int32)
    src_p = jnp.concatenate([src, pad % N])
    dst_p = jnp.concatenate([dst, N + (pad % (ACC_ROWS - N))])
    src2 = src_p.reshape(NT, NB2, B2)
    dst_prop = dst_p.reshape(NT, NB2, B2)
    dst_cnt = dst_p.reshape(NC, NT, NB_CNT, B)

    cnt = _sc_count(dst_cnt)
    xp = _prep_call(x, cnt)
    p1 = _sc_prop(xp.reshape(NC * N, 128), src2, dst_prop)
    t2 = _mm1_call(p1, cnt, W1, b1.reshape(1, D_HID), W2)
    p2 = _sc_prop(t2.reshape(NC * N, 128), src2, dst_prop)
    out = _fin_call(p2, cnt, b2.reshape(1, D_HID2), Wout, bout.reshape(1, 1))
    return out


# speedup vs baseline: 1.0238x; 1.0238x over previous
import functools

import jax
import jax.numpy as jnp
from jax import lax
from jax.experimental import pallas as pl
from jax.experimental.pallas import tpu as pltpu
from jax.experimental.pallas import tpu_sc as plsc

N = 10000
E = 160000
D_IN = 256
D_HID = 512
D_HID2 = 256
NC = 2
NT = 16
B = 128
EP = NC * NT * 40 * B
NB_PROP = EP // (NT * B)
NB_DEG = EP // (NC * NT * B)
ACC_ROWS = N + 16
S0 = 632
SLAST = N - (NT - 1) * S0
BM = 2000

_mesh = plsc.VectorSubcoreMesh(core_axis_name="c", subcore_axis_name="s")


B2 = 64
NB2 = EP // (NT * B2)
CB = 40
NCHUNK = NB2 // CB
NBUF = 4


@functools.partial(
    pl.kernel,
    out_type=jax.ShapeDtypeStruct((NC, N, 128), jnp.float32),
    mesh=_mesh,
    scratch_types=[
        pltpu.VMEM_SHARED((ACC_ROWS, 128), jnp.float32),
        pltpu.VMEM((CB, B2), jnp.int32),
        pltpu.VMEM((CB, B2), jnp.int32),
        pltpu.VMEM((NBUF, B2, 128), jnp.float32),
        pltpu.SemaphoreType.DMA((NBUF,)),
    ],
)
def _sc_prop(table_hbm, src_hbm, dstidx_hbm, out_hbm,
             acc, src_v, dst_v, bufs, sems):
    c = lax.axis_index("c")
    s = lax.axis_index("s")

    @pl.when(s < NT - 1)
    def _():
        pltpu.sync_copy(table_hbm.at[pl.ds(c * N + s * S0, S0)],
                        acc.at[pl.ds(s * S0, S0)])

    @pl.when(s == NT - 1)
    def _():
        pltpu.sync_copy(table_hbm.at[pl.ds(c * N + (NT - 1) * S0, SLAST)],
                        acc.at[pl.ds((NT - 1) * S0, SLAST)])

    plsc.subcore_barrier()

    def gather(b, j):
        pltpu.async_copy(table_hbm.at[src_v.at[b]], bufs.at[j], sems.at[j])

    coff = (c * N).astype(jnp.int32)

    def chunk(k, _):
        pltpu.sync_copy(src_hbm.at[s, pl.ds(k * CB, CB)], src_v)
        pltpu.sync_copy(dstidx_hbm.at[s, pl.ds(k * CB, CB)], dst_v)

        def shift(i, _):
            r = i // 4
            q = (i % 4) * 16
            src_v[r, pl.ds(q, 16)] = src_v[r, pl.ds(q, 16)] + coff
            return 0

        lax.fori_loop(0, CB * 4, shift, 0)
        for j in range(NBUF - 1):
            gather(j, j)

        def group(g, _):
            for j in range(NBUF):
                b = NBUF * g + j
                pltpu.make_async_copy(table_hbm.at[src_v.at[b]],
                                      bufs.at[j], sems.at[j]).wait()
                pltpu.sync_copy(bufs.at[j], acc.at[dst_v.at[b]], add=True)

                @pl.when(b + NBUF - 1 < CB)
                def _(b=b, j=j):
                    gather(b + NBUF - 1, (j + NBUF - 1) % NBUF)
            return 0

        lax.fori_loop(0, CB // NBUF, group, 0)
        return 0

    lax.fori_loop(0, NCHUNK, chunk, 0)
    plsc.subcore_barrier()

    @pl.when(s < NT - 1)
    def _():
        pltpu.sync_copy(acc.at[pl.ds(s * S0, S0)],
                        out_hbm.at[c, pl.ds(s * S0, S0)])

    @pl.when(s == NT - 1)
    def _():
        pltpu.sync_copy(acc.at[pl.ds((NT - 1) * S0, SLAST)],
                        out_hbm.at[c, pl.ds((NT - 1) * S0, SLAST)])


NB_CNT = EP // (NC * NT * B)
_Z_CHUNKS = ((128, 128, 128, 128, 120), (128, 128, 128, 128, 8))


@functools.partial(
    pl.kernel,
    out_type=jax.ShapeDtypeStruct((NC, N, 128), jnp.float32),
    mesh=_mesh,
    scratch_types=[
        pltpu.VMEM_SHARED((ACC_ROWS, 128), jnp.float32),
        pltpu.VMEM((NB_CNT, B), jnp.int32),
        pltpu.VMEM((B, 128), jnp.float32),
        pltpu.VMEM((B, 128), jnp.float32),
    ],
)
def _sc_count(dst_hbm, cnt_hbm, acc, dst_v, ones_v, z_v):
    c = lax.axis_index("c")
    s = lax.axis_index("s")
    pltpu.sync_copy(dst_hbm.at[c, s], dst_v)

    def fill(i, _):
        r = i // 8
        j = i % 8
        ones_v[r, pl.ds(j * 16, 16)] = jnp.ones((16,), jnp.float32)
        z_v[r, pl.ds(j * 16, 16)] = jnp.zeros((16,), jnp.float32)
        return 0

    lax.fori_loop(0, B * 8, fill, 0)

    for last, sizes in ((False, _Z_CHUNKS[0]), (True, _Z_CHUNKS[1])):

        @pl.when((s == NT - 1) if last else (s < NT - 1))
        def _(sizes=sizes):
            off = 0
            for sz in sizes:
                pltpu.sync_copy(z_v.at[pl.ds(0, sz)],
                                acc.at[pl.ds(s * S0 + off, sz)])
                off += sz

    plsc.subcore_barrier()

    def body(b, _):
        pltpu.sync_copy(ones_v, acc.at[dst_v.at[b]], add=True)
        return 0

    lax.fori_loop(0, NB_CNT, body, 0)
    plsc.subcore_barrier()

    @pl.when(s < NT - 1)
    def _():
        pltpu.sync_copy(acc.at[pl.ds(s * S0, S0)],
                        cnt_hbm.at[c, pl.ds(s * S0, S0)])

    @pl.when(s == NT - 1)
    def _():
        pltpu.sync_copy(acc.at[pl.ds((NT - 1) * S0, SLAST)],
                        cnt_hbm.at[c, pl.ds((NT - 1) * S0, SLAST)])


def _prep_body(x_ref, cnt_ref, o_ref):
    d = lax.rsqrt(cnt_ref[0] + cnt_ref[1] + 1.0)
    o_ref[0] = x_ref[...] * d


_prep_call = pl.pallas_call(
    _prep_body,
    grid=(2,),
    in_specs=[
        pl.BlockSpec((N, 128), lambda c: (0, c)),
        pl.BlockSpec((NC, N, 128), lambda c: (0, 0, 0)),
    ],
    out_specs=pl.BlockSpec((1, N, 128), lambda c: (c, 0, 0)),
    out_shape=jax.ShapeDtypeStruct((NC, N, 128), jnp.float32),
)


def _mm1_body(p_ref, cnt_ref, w1_ref, b1_ref, w2_ref, o_ref):
    d = lax.rsqrt(cnt_ref[0] + cnt_ref[1] + 1.0)
    p0 = p_ref[0] * d
    p1 = p_ref[1] * d
    h = jnp.dot(p0, w1_ref[:128, :], preferred_element_type=jnp.float32)
    h = h + jnp.dot(p1, w1_ref[128:, :], preferred_element_type=jnp.float32)
    h = jnp.maximum(h + b1_ref[...], 0.0)
    t = jnp.dot(h, w2_ref[...], preferred_element_type=jnp.float32)
    o_ref[0] = t[:, :128] * d
    o_ref[1] = t[:, 128:] * d


_mm1_call = pl.pallas_call(
    _mm1_body,
    grid=(N // BM,),
    in_specs=[
        pl.BlockSpec((NC, BM, 128), lambda i: (0, i, 0)),
        pl.BlockSpec((NC, BM, 128), lambda i: (0, i, 0)),
        pl.BlockSpec((D_IN, D_HID), lambda i: (0, 0)),
        pl.BlockSpec((1, D_HID), lambda i: (0, 0)),
        pl.BlockSpec((D_HID, D_HID2), lambda i: (0, 0)),
    ],
    out_specs=pl.BlockSpec((NC, BM, 128), lambda i: (0, i, 0)),
    out_shape=jax.ShapeDtypeStruct((NC, N, 128), jnp.float32),
)


def _fin_body(p_ref, cnt_ref, b2_ref, wo_ref, bo_ref, o_ref):
    d = lax.rsqrt(cnt_ref[0] + cnt_ref[1] + 1.0)
    b2 = b2_ref[...]
    h0 = jnp.maximum(p_ref[0] * d + b2[:, :128], 0.0)
    h1 = jnp.maximum(p_ref[1] * d + b2[:, 128:], 0.0)
    o = jnp.dot(h0, wo_ref[:128, :], preferred_element_type=jnp.float32)
    o = o + jnp.dot(h1, wo_ref[128:, :], preferred_element_type=jnp.float32)
    o_ref[...] = o + bo_ref[...]


_fin_call = pl.pallas_call(
    _fin_body,
    grid=(N // BM,),
    in_specs=[
        pl.BlockSpec((NC, BM, 128), lambda i: (0, i, 0)),
        pl.BlockSpec((NC, BM, 128), lambda i: (0, i, 0)),
        pl.BlockSpec((1, D_HID2), lambda i: (0, 0)),
        pl.BlockSpec((D_HID2, 1), lambda i: (0, 0)),
        pl.BlockSpec((1, 1), lambda i: (0, 0)),
    ],
    out_specs=pl.BlockSpec((BM, 1), lambda i: (i, 0)),
    out_shape=jax.ShapeDtypeStruct((N, 1), jnp.float32),
)


def kernel(x, edge_index, W1, b1, W2, b2, Wout, bout):
    src = edge_index[0].astype(jnp.int32)
    dst = edge_index[1].astype(jnp.int32)
    pad = jnp.arange(EP - E, dtype=jnp.int32)
    src_p = jnp.concatenate([src, pad % N])
    dst_p = jnp.concatenate([dst, N + (pad % (ACC_ROWS - N))])
    src2 = src_p.reshape(NT, NB2, B2)
    dst_prop = dst_p.reshape(NT, NB2, B2)
    dst_cnt = dst_p.reshape(NC, NT, NB_CNT, B)

    cnt = _sc_count(dst_cnt)
    xp = _prep_call(x, cnt)
    p1 = _sc_prop(xp.reshape(NC * N, 128), src2, dst_prop)
    t2 = _mm1_call(p1, cnt, W1, b1.reshape(1, D_HID), W2)
    p2 = _sc_prop(t2.reshape(NC * N, 128), src2, dst_prop)
    out = _fin_call(p2, cnt, b2.reshape(1, D_HID2), Wout, bout.reshape(1, 1))
    return out
